# Initial kernel scaffold; baseline (speedup 1.0000x reference)
import jax
import jax.numpy as jnp
from jax.experimental import pallas as pl


def kernel(x_2, incidence_2, w1, w2):
    raise NotImplementedError("write your pallas kernel here")


# speedup vs baseline: 1.2454x; 1.2454x over previous
import jax
import jax.numpy as jnp
from jax.experimental import pallas as pl
from jax.experimental.pallas import tpu as pltpu


def _pass1_body(x2_ref, w1_ref, inc_ref, x1_ref, m1_ref):
    @pl.when(pl.program_id(0) == 0)
    def _():
        m1_ref[...] = jnp.dot(
            x2_ref[...], w1_ref[...], preferred_element_type=jnp.float32
        )

    blk = inc_ref[...]
    y = jnp.dot(blk, m1_ref[...], preferred_element_type=jnp.float32)
    rs = jnp.sum(blk, axis=1, keepdims=True)
    x1_ref[...] = jax.nn.sigmoid(y * (1.0 / rs))


def _pass2_body(x1_ref, w2_ref, inc_ref, out_ref, m2_ref):
    @pl.when(pl.program_id(0) == 0)
    def _():
        m2_ref[...] = jnp.dot(
            x1_ref[...], w2_ref[...], preferred_element_type=jnp.float32
        )

    blk = inc_ref[...]
    y = jax.lax.dot_general(
        blk,
        m2_ref[...],
        (((0,), (0,)), ((), ())),
        preferred_element_type=jnp.float32,
    )
    cs = jnp.sum(blk, axis=0)[:, None]
    out_ref[...] = jax.nn.sigmoid(y * (1.0 / cs))


def kernel(x_2, incidence_2, w1, w2):
    n_edges, n_faces = incidence_2.shape
    in_c = x_2.shape[1]
    mid_c = w1.shape[1]
    out_c = w2.shape[1]
    bm = 512
    bn = 512

    x_1 = pl.pallas_call(
        _pass1_body,
        grid=(n_edges // bm,),
        in_specs=[
            pl.BlockSpec((n_faces, in_c), lambda i: (0, 0)),
            pl.BlockSpec((in_c, mid_c), lambda i: (0, 0)),
            pl.BlockSpec((bm, n_faces), lambda i: (i, 0)),
        ],
        out_specs=pl.BlockSpec((bm, mid_c), lambda i: (i, 0)),
        out_shape=jax.ShapeDtypeStruct((n_edges, mid_c), jnp.float32),
        scratch_shapes=[pltpu.VMEM((n_faces, mid_c), jnp.float32)],
    )(x_2, w1, incidence_2)

    out = pl.pallas_call(
        _pass2_body,
        grid=(n_faces // bn,),
        in_specs=[
            pl.BlockSpec((n_edges, mid_c), lambda j: (0, 0)),
            pl.BlockSpec((mid_c, out_c), lambda j: (0, 0)),
            pl.BlockSpec((n_edges, bn), lambda j: (0, j)),
        ],
        out_specs=pl.BlockSpec((bn, out_c), lambda j: (j, 0)),
        out_shape=jax.ShapeDtypeStruct((n_faces, out_c), jnp.float32),
        scratch_shapes=[pltpu.VMEM((n_edges, out_c), jnp.float32)],
    )(x_1, w2, incidence_2)

    return out


# speedup vs baseline: 1.2970x; 1.0415x over previous
import jax
import jax.numpy as jnp
from jax.experimental import pallas as pl
from jax.experimental.pallas import tpu as pltpu


def _pass1_body(x2_ref, w1_ref, inc_ref, x1_ref, m1_ref):
    @pl.when(pl.program_id(0) == 0)
    def _():
        m1_ref[...] = jnp.dot(
            x2_ref[...], w1_ref[...], preferred_element_type=jnp.float32
        ).astype(jnp.bfloat16)

    blk = inc_ref[...]
    y = jnp.dot(
        blk.astype(jnp.bfloat16), m1_ref[...], preferred_element_type=jnp.float32
    )
    rs = jnp.sum(blk, axis=1, keepdims=True)
    x1_ref[...] = jax.nn.sigmoid(y * (1.0 / rs))


def _pass2_body(x1_ref, w2_ref, inc_ref, out_ref, m2_ref):
    @pl.when(pl.program_id(0) == 0)
    def _():
        m2_ref[...] = jnp.dot(
            x1_ref[...], w2_ref[...], preferred_element_type=jnp.float32
        ).astype(jnp.bfloat16)

    blk = inc_ref[...]
    y = jax.lax.dot_general(
        blk.astype(jnp.bfloat16),
        m2_ref[...],
        (((0,), (0,)), ((), ())),
        preferred_element_type=jnp.float32,
    )
    cs = jnp.sum(blk, axis=0)[:, None]
    out_ref[...] = jax.nn.sigmoid(y * (1.0 / cs))


def kernel(x_2, incidence_2, w1, w2):
    n_edges, n_faces = incidence_2.shape
    in_c = x_2.shape[1]
    mid_c = w1.shape[1]
    out_c = w2.shape[1]
    bm = 512
    bn = 512

    x_1 = pl.pallas_call(
        _pass1_body,
        grid=(n_edges // bm,),
        in_specs=[
            pl.BlockSpec((n_faces, in_c), lambda i: (0, 0)),
            pl.BlockSpec((in_c, mid_c), lambda i: (0, 0)),
            pl.BlockSpec((bm, n_faces), lambda i: (i, 0)),
        ],
        out_specs=pl.BlockSpec((bm, mid_c), lambda i: (i, 0)),
        out_shape=jax.ShapeDtypeStruct((n_edges, mid_c), jnp.float32),
        scratch_shapes=[pltpu.VMEM((n_faces, mid_c), jnp.bfloat16)],
    )(x_2, w1, incidence_2)

    out = pl.pallas_call(
        _pass2_body,
        grid=(n_faces // bn,),
        in_specs=[
            pl.BlockSpec((n_edges, mid_c), lambda j: (0, 0)),
            pl.BlockSpec((mid_c, out_c), lambda j: (0, 0)),
            pl.BlockSpec((n_edges, bn), lambda j: (0, j)),
        ],
        out_specs=pl.BlockSpec((bn, out_c), lambda j: (j, 0)),
        out_shape=jax.ShapeDtypeStruct((n_faces, out_c), jnp.float32),
        scratch_shapes=[pltpu.VMEM((n_edges, out_c), jnp.bfloat16)],
    )(x_1, w2, incidence_2)

    return out


# speedup vs baseline: 1.3364x; 1.0304x over previous
import jax
import jax.numpy as jnp
from jax.experimental import pallas as pl
from jax.experimental.pallas import tpu as pltpu


def _body(x2_ref, w1p_ref, w2p_ref, inc_ref, out_ref, m1e_ref, acc_ref):
    i = pl.program_id(0)
    nsteps = pl.num_programs(0)
    bm = inc_ref.shape[0]
    n_faces = inc_ref.shape[1]

    @pl.when(i == 0)
    def _():
        m1p = jnp.dot(x2_ref[...], w1p_ref[...], preferred_element_type=jnp.float32)
        col = jax.lax.broadcasted_iota(jnp.int32, m1p.shape, 1)
        m1e_ref[...] = jnp.where(col == 64, 1.0, m1p).astype(jnp.bfloat16)

    blk = inc_ref[...].astype(jnp.bfloat16)
    y1e = jnp.dot(blk, m1e_ref[...], preferred_element_type=jnp.float32)
    y1 = y1e[:, :64]
    rs = y1e[:, 64:65]
    x1_blk = jax.nn.sigmoid(y1 * (1.0 / rs))

    m2p = jnp.dot(x1_blk, w2p_ref[...], preferred_element_type=jnp.float32)
    col = jax.lax.broadcasted_iota(jnp.int32, m2p.shape, 1)
    m2e = jnp.where(col == 64, 1.0, m2p).astype(jnp.bfloat16)

    contrib = jax.lax.dot_general(
        blk, m2e, (((0,), (0,)), ((), ())), preferred_element_type=jnp.float32
    )

    @pl.when(i == 0)
    def _():
        acc_ref[...] = contrib

    @pl.when(i > 0)
    def _():
        acc_ref[...] += contrib

    @pl.when(i == nsteps - 1)
    def _():
        y2 = acc_ref[:, :64]
        cs = acc_ref[:, 64:65]
        out_ref[...] = jax.nn.sigmoid(y2 * (1.0 / cs))


def kernel(x_2, incidence_2, w1, w2):
    n_edges, n_faces = incidence_2.shape
    in_c = x_2.shape[1]
    mid_c = w1.shape[1]
    out_c = w2.shape[1]
    bm = 512

    w1p = jnp.pad(w1, ((0, 0), (0, 128 - mid_c)))
    w2p = jnp.pad(w2, ((0, 0), (0, 128 - out_c)))

    out = pl.pallas_call(
        _body,
        grid=(n_edges // bm,),
        in_specs=[
            pl.BlockSpec((n_faces, in_c), lambda i: (0, 0)),
            pl.BlockSpec((in_c, 128), lambda i: (0, 0)),
            pl.BlockSpec((mid_c, 128), lambda i: (0, 0)),
            pl.BlockSpec((bm, n_faces), lambda i: (i, 0)),
        ],
        out_specs=pl.BlockSpec((n_faces, out_c), lambda i: (0, 0)),
        out_shape=jax.ShapeDtypeStruct((n_faces, out_c), jnp.float32),
        scratch_shapes=[
            pltpu.VMEM((n_faces, 128), jnp.bfloat16),
            pltpu.VMEM((n_faces, 128), jnp.float32),
        ],
    )(x_2, w1p, w2p, incidence_2)

    return out


# speedup vs baseline: 1.7221x; 1.2886x over previous
import jax
import jax.numpy as jnp
from jax.experimental import pallas as pl
from jax.experimental.pallas import tpu as pltpu


def _body(x2_ref, w1p_ref, w2p_ref, inc_ref, out_ref, m1e_ref, acc_ref):
    i = pl.program_id(0)
    nsteps = pl.num_programs(0)
    bm = inc_ref.shape[0]
    n_faces = inc_ref.shape[1]

    @pl.when(i == 0)
    def _():
        m1p = jnp.dot(x2_ref[...], w1p_ref[...], preferred_element_type=jnp.float32)
        col = jax.lax.broadcasted_iota(jnp.int32, m1p.shape, 1)
        m1e_ref[...] = jnp.where(col == 64, 1.0, m1p).astype(jnp.bfloat16)

    blk = inc_ref[...].astype(jnp.bfloat16)
    y1e = jnp.dot(blk, m1e_ref[...], preferred_element_type=jnp.float32)
    y1 = y1e[:, :64]
    rs = y1e[:, 64:65]
    x1_blk = jax.nn.sigmoid(y1 * (1.0 / rs))

    m2p = jnp.dot(x1_blk, w2p_ref[...], preferred_element_type=jnp.float32)
    col = jax.lax.broadcasted_iota(jnp.int32, m2p.shape, 1)
    m2e = jnp.where(col == 64, 1.0, m2p).astype(jnp.bfloat16)

    contrib = jax.lax.dot_general(
        m2e, blk, (((0,), (0,)), ((), ())), preferred_element_type=jnp.float32
    )

    @pl.when(i == 0)
    def _():
        acc_ref[...] = contrib

    @pl.when(i > 0)
    def _():
        acc_ref[...] += contrib

    @pl.when(i == nsteps - 1)
    def _():
        y2 = acc_ref[:64, :]
        cs = acc_ref[64:65, :]
        out_ref[...] = jnp.transpose(jax.nn.sigmoid(y2 * (1.0 / cs)))


def kernel(x_2, incidence_2, w1, w2):
    n_edges, n_faces = incidence_2.shape
    in_c = x_2.shape[1]
    mid_c = w1.shape[1]
    out_c = w2.shape[1]
    bm = 512

    w1p = jnp.pad(w1, ((0, 0), (0, 128 - mid_c)))
    w2p = jnp.pad(w2, ((0, 0), (0, 128 - out_c)))

    out = pl.pallas_call(
        _body,
        grid=(n_edges // bm,),
        in_specs=[
            pl.BlockSpec((n_faces, in_c), lambda i: (0, 0)),
            pl.BlockSpec((in_c, 128), lambda i: (0, 0)),
            pl.BlockSpec((mid_c, 128), lambda i: (0, 0)),
            pl.BlockSpec((bm, n_faces), lambda i: (i, 0)),
        ],
        out_specs=pl.BlockSpec((n_faces, out_c), lambda i: (0, 0)),
        out_shape=jax.ShapeDtypeStruct((n_faces, out_c), jnp.float32),
        scratch_shapes=[
            pltpu.VMEM((n_faces, 128), jnp.bfloat16),
            pltpu.VMEM((128, n_faces), jnp.float32),
        ],
    )(x_2, w1p, w2p, incidence_2)

    return out
